# baseline (speedup 1.0000x reference)
import functools

import jax
import jax.numpy as jnp
from jax import lax
from jax.experimental import pallas as pl
from jax.experimental.pallas import tpu as pltpu
from jax.experimental.pallas import tpu_sc as plsc

_B, _S, _D = 4, 8192, 1024
_NW = 32
_SW = _S // _NW
_RC = 8
_NCH = _SW // _RC
_KV = _D // 16


def _sc_body(x_hbm, t_hbm, o_hbm,
             x0, x1, x2, t0, t1, t2,
             si0, si1, si2, st0, st1, st2, so0, so1, so2):
    cid = lax.axis_index("c")
    sid = lax.axis_index("s")
    wid = sid * 2 + cid
    s0 = wid * _SW

    xbufs = (x0, x1, x2)
    tbufs = (t0, t1, t2)
    sin = (si0, si1, si2)
    sts = (st0, st1, st2)
    sout = (so0, so1, so2)

    def in_copy(c, n, b):
        return pltpu.make_async_copy(
            x_hbm.at[b, pl.ds(s0 + c * _RC, _RC), :], xbufs[n].at[b], sin[n])

    def out_copy(c, n, b):
        return pltpu.make_async_copy(
            xbufs[n].at[b], o_hbm.at[b, pl.ds(s0 + c * _RC, _RC), :], sout[n])

    def t_copy(c, n):
        return pltpu.make_async_copy(
            t_hbm.at[pl.ds(s0 + c * _RC, _RC), :], tbufs[n], sts[n])

    def start_chunk(c, n):
        t_copy(c, n).start()
        for b in range(_B):
            in_copy(c, n, b).start()

    def compute(n):
        xbuf, tbuf = xbufs[n], tbufs[n]

        @plsc.parallel_loop(0, _KV, unroll=2)
        def _(k):
            sl = pl.ds(k * 16, 16)
            for i in range(_RC):
                tv = tbuf[i, sl]
                for b in range(_B):
                    xbuf[b, i, sl] = xbuf[b, i, sl] + tv

    def phase(p, n, refill, guard):
        t_copy(p, n).wait()
        for b in range(_B):
            in_copy(p, n, b).wait()
        compute(n)
        for b in range(_B):
            out_copy(p, n, b).start()

        def do_refill():
            m = (n + 1) % 3
            for b in range(_B):
                out_copy(p - 2, m, b).wait()
            start_chunk(p + 1, m)

        if refill:
            if guard:
                pl.when(p >= 2)(do_refill)
            else:
                do_refill()

    start_chunk(0, 0)
    start_chunk(1, 1)
    start_chunk(2, 2)

    def iteration(g, carry):
        p = 3 * g
        phase(p, 0, refill=True, guard=True)
        phase(p + 1, 1, refill=True, guard=True)
        phase(p + 2, 2, refill=True, guard=False)
        return carry

    lax.fori_loop(0, _NCH // 3, iteration, 0)

    phase(30, 0, refill=True, guard=False)
    phase(31, 1, refill=False, guard=False)
    for b in range(_B):
        out_copy(29, 2, b).wait()
    for b in range(_B):
        out_copy(30, 0, b).wait()
    for b in range(_B):
        out_copy(31, 1, b).wait()


def kernel(x, table):
    mesh = plsc.VectorSubcoreMesh(core_axis_name="c", subcore_axis_name="s")
    dma = pltpu.SemaphoreType.DMA
    k = functools.partial(
        pl.kernel,
        mesh=mesh,
        out_type=jax.ShapeDtypeStruct((_B, _S, _D), jnp.float32),
        scratch_types=[
            pltpu.VMEM((_B, _RC, _D), jnp.float32),
            pltpu.VMEM((_B, _RC, _D), jnp.float32),
            pltpu.VMEM((_B, _RC, _D), jnp.float32),
            pltpu.VMEM((_RC, _D), jnp.float32),
            pltpu.VMEM((_RC, _D), jnp.float32),
            pltpu.VMEM((_RC, _D), jnp.float32),
            dma, dma, dma, dma, dma, dma, dma, dma, dma,
        ],
    )(_sc_body)
    return k(x, table)


# speedup vs baseline: 1.1835x; 1.1835x over previous
import functools

import jax
import jax.numpy as jnp
from jax import lax
from jax.experimental import pallas as pl
from jax.experimental.pallas import tpu as pltpu
from jax.experimental.pallas import tpu_sc as plsc

_B, _S, _D = 4, 8192, 1024
_NW = 32
_SW = _S // _NW
_RC = 8
_NCH = _SW // _RC
_KV = _D // 16


def _sc_body(x_hbm, t_hbm, o_hbm, xA, xB, tA, tB, siA, siB, stA, stB,
             soA, soB):
    cid = lax.axis_index("c")
    sid = lax.axis_index("s")
    wid = sid * 2 + cid
    s0 = wid * _SW

    def in_copy(c, xbuf, b, sem):
        return pltpu.make_async_copy(
            x_hbm.at[b, pl.ds(s0 + c * _RC, _RC), :], xbuf.at[b], sem)

    def out_copy(c, xbuf, b, sem):
        return pltpu.make_async_copy(
            xbuf.at[b], o_hbm.at[b, pl.ds(s0 + c * _RC, _RC), :], sem)

    def t_copy(c, tbuf, sem):
        return pltpu.make_async_copy(
            t_hbm.at[pl.ds(s0 + c * _RC, _RC), :], tbuf, sem)

    def start_chunk(c, xbuf, tbuf, sem_in, sem_t):
        t_copy(c, tbuf, sem_t).start()
        for b in range(_B):
            in_copy(c, xbuf, b, sem_in).start()

    def wait_chunk_in(c, xbuf, tbuf, sem_in, sem_t):
        t_copy(c, tbuf, sem_t).wait()
        for b in range(_B):
            in_copy(c, xbuf, b, sem_in).wait()

    def compute(xbuf, tbuf):
        @plsc.parallel_loop(0, _KV, unroll=3)
        def _(k):
            sl = pl.ds(k * 16, 16)
            for i in range(_RC):
                tv = tbuf[i, sl]
                for b in range(_B):
                    xbuf[b, i, sl] = xbuf[b, i, sl] + tv

    start_chunk(0, xA, tA, siA, stA)
    start_chunk(1, xB, tB, siB, stB)

    def iteration(g, carry):
        c0 = 2 * g
        c1 = c0 + 1
        wait_chunk_in(c0, xA, tA, siA, stA)
        compute(xA, tA)
        for b in range(_B):
            out_copy(c0, xA, b, soA).start()
        wait_chunk_in(c1, xB, tB, siB, stB)
        compute(xB, tB)
        for b in range(_B):
            out_copy(c1, xB, b, soB).start()
        for b in range(_B):
            out_copy(c0, xA, b, soA).wait()

        @pl.when(c0 + 2 < _NCH)
        def _():
            start_chunk(c0 + 2, xA, tA, siA, stA)

        for b in range(_B):
            out_copy(c1, xB, b, soB).wait()

        @pl.when(c1 + 2 < _NCH)
        def _():
            start_chunk(c1 + 2, xB, tB, siB, stB)

        return carry

    lax.fori_loop(0, _NCH // 2, iteration, 0)


def kernel(x, table):
    mesh = plsc.VectorSubcoreMesh(core_axis_name="c", subcore_axis_name="s")
    k = functools.partial(
        pl.kernel,
        mesh=mesh,
        out_type=jax.ShapeDtypeStruct((_B, _S, _D), jnp.float32),
        scratch_types=[
            pltpu.VMEM((_B, _RC, _D), jnp.float32),
            pltpu.VMEM((_B, _RC, _D), jnp.float32),
            pltpu.VMEM((_RC, _D), jnp.float32),
            pltpu.VMEM((_RC, _D), jnp.float32),
            pltpu.SemaphoreType.DMA,
            pltpu.SemaphoreType.DMA,
            pltpu.SemaphoreType.DMA,
            pltpu.SemaphoreType.DMA,
            pltpu.SemaphoreType.DMA,
            pltpu.SemaphoreType.DMA,
        ],
    )(_sc_body)
    return k(x, table)


# speedup vs baseline: 1.2621x; 1.0665x over previous
import functools

import jax
import jax.numpy as jnp
from jax import lax
from jax.experimental import pallas as pl
from jax.experimental.pallas import tpu as pltpu
from jax.experimental.pallas import tpu_sc as plsc

_B, _S, _D = 4, 8192, 1024
_NW = 32
_SW = _S // _NW
_RC = 8
_NCH = _SW // _RC
_KV = _D // 16


def _sc_body(x_hbm, t_hbm, o_hbm, xA, xB, tA, tB, siA, siB, stA, stB,
             soA, soB):
    cid = lax.axis_index("c")
    sid = lax.axis_index("s")
    wid = sid * 2 + cid
    s0 = wid * _SW

    def in_copy(c, xbuf, b, sem):
        return pltpu.make_async_copy(
            x_hbm.at[b, pl.ds(s0 + c * _RC, _RC), :], xbuf.at[b], sem)

    def out_copy(c, xbuf, b, sem):
        return pltpu.make_async_copy(
            xbuf.at[b], o_hbm.at[b, pl.ds(s0 + c * _RC, _RC), :], sem)

    def t_copy(c, tbuf, sem):
        return pltpu.make_async_copy(
            t_hbm.at[pl.ds(s0 + c * _RC, _RC), :], tbuf, sem)

    def start_chunk(c, xbuf, tbuf, sem_in, sem_t):
        t_copy(c, tbuf, sem_t).start()
        for b in range(_B):
            in_copy(c, xbuf, b, sem_in).start()

    def wait_chunk_in(c, xbuf, tbuf, sem_in, sem_t):
        t_copy(c, tbuf, sem_t).wait()
        for b in range(_B):
            in_copy(c, xbuf, b, sem_in).wait()

    def compute(xbuf, tbuf):
        @plsc.parallel_loop(0, _KV, unroll=2)
        def _(k):
            sl = pl.ds(k * 16, 16)
            for i in range(_RC):
                tv = tbuf[i, sl]
                for b in range(_B):
                    xbuf[b, i, sl] = xbuf[b, i, sl] + tv

    start_chunk(0, xA, tA, siA, stA)
    start_chunk(1, xB, tB, siB, stB)

    def iteration(g, carry):
        c0 = 2 * g
        c1 = c0 + 1
        wait_chunk_in(c0, xA, tA, siA, stA)
        compute(xA, tA)
        for b in range(_B):
            out_copy(c0, xA, b, soA).start()
        wait_chunk_in(c1, xB, tB, siB, stB)
        compute(xB, tB)
        for b in range(_B):
            out_copy(c1, xB, b, soB).start()
        for b in range(_B):
            out_copy(c0, xA, b, soA).wait()

        @pl.when(c0 + 2 < _NCH)
        def _():
            start_chunk(c0 + 2, xA, tA, siA, stA)

        for b in range(_B):
            out_copy(c1, xB, b, soB).wait()

        @pl.when(c1 + 2 < _NCH)
        def _():
            start_chunk(c1 + 2, xB, tB, siB, stB)

        return carry

    lax.fori_loop(0, _NCH // 2, iteration, 0)


def kernel(x, table):
    mesh = plsc.VectorSubcoreMesh(core_axis_name="c", subcore_axis_name="s")
    k = functools.partial(
        pl.kernel,
        mesh=mesh,
        out_type=jax.ShapeDtypeStruct((_B, _S, _D), jnp.float32),
        scratch_types=[
            pltpu.VMEM((_B, _RC, _D), jnp.float32),
            pltpu.VMEM((_B, _RC, _D), jnp.float32),
            pltpu.VMEM((_RC, _D), jnp.float32),
            pltpu.VMEM((_RC, _D), jnp.float32),
            pltpu.SemaphoreType.DMA,
            pltpu.SemaphoreType.DMA,
            pltpu.SemaphoreType.DMA,
            pltpu.SemaphoreType.DMA,
            pltpu.SemaphoreType.DMA,
            pltpu.SemaphoreType.DMA,
        ],
    )(_sc_body)
    return k(x, table)
